# Initial kernel scaffold; baseline (speedup 1.0000x reference)
import functools

import jax
import jax.numpy as jnp
from jax import lax
from jax.experimental import pallas as pl
from jax.experimental.pallas import tpu as pltpu
from jax.experimental.pallas import tpu_sc as plsc

N = 256
_INFO = plsc.get_sparse_core_info()
NC, NS, L = _INFO.num_cores, _INFO.num_subcores, _INFO.num_lanes
NW = NC * NS
RPW = N // NW
CHUNK = RPW * N
NCOL = N // L

_SQRT2 = 1.4142135623730951
_C1 = 2.885390081777927
_C3 = 0.9617966939259756
_C5 = 0.5770780163555854
_C7 = 0.4121985945111324
_C9 = 0.3205988979753252


def _log2(u):
    bits = lax.bitcast_convert_type(u, jnp.int32)
    e = (bits >> 23) - 127
    m = lax.bitcast_convert_type((bits & 0x7FFFFF) | 0x3F800000, jnp.float32)
    big = m >= jnp.float32(_SQRT2)
    m2 = jnp.where(big, m * jnp.float32(0.5), m)
    ef = e.astype(jnp.float32) + jnp.where(big, jnp.float32(1.0), jnp.float32(0.0))
    z = (m2 - jnp.float32(1.0)) / (m2 + jnp.float32(1.0))
    z2 = z * z
    pz = z * (jnp.float32(_C1) + z2 * (jnp.float32(_C3) + z2 * (
        jnp.float32(_C5) + z2 * (jnp.float32(_C7) + z2 * jnp.float32(_C9)))))
    return ef + pz


def _bit(p, u0, u1):
    w0 = -_log2(u0)
    w1 = -_log2(u1)
    t = jnp.exp(jnp.float32(2.0) * p - jnp.float32(1.0))
    return jnp.where(w1 * t >= w0, jnp.float32(1.0), jnp.float32(0.0))


def _body(rp_h, ur0_h, ur1_h, ep_h, u0_h, u1_h, ept_h, u0t_h, u1t_h, out_h,
          rp_v, ur0_v, ur1_v, croot_v,
          ep_v, u0_v, u1_v, ept_v, u0t_v, u1t_v, out_v):
    wid = lax.axis_index("s") * NC + lax.axis_index("c")
    base = wid * CHUNK

    pltpu.sync_copy(rp_h, rp_v)
    pltpu.sync_copy(ur0_h, ur0_v)
    pltpu.sync_copy(ur1_h, ur1_v)
    pltpu.sync_copy(ep_h.at[pl.ds(base, CHUNK)], ep_v)
    pltpu.sync_copy(u0_h.at[pl.ds(base, CHUNK)], u0_v)
    pltpu.sync_copy(u1_h.at[pl.ds(base, CHUNK)], u1_v)
    pltpu.sync_copy(ept_h.at[pl.ds(base, CHUNK)], ept_v)
    pltpu.sync_copy(u0t_h.at[pl.ds(base, CHUNK)], u0t_v)
    pltpu.sync_copy(u1t_h.at[pl.ds(base, CHUNK)], u1t_v)

    def croot_body(c, carry):
        s = c * L
        bit = _bit(rp_v[pl.ds(s, L)], ur0_v[pl.ds(s, L)], ur1_v[pl.ds(s, L)])
        croot_v[pl.ds(s, L)] = jnp.float32(1.0) - bit
        return carry

    lax.fori_loop(0, NCOL, croot_body, 0)

    lanes = lax.iota(jnp.int32, L)
    for li in range(RPW):
        i = wid * RPW + li
        ci = plsc.load_gather(croot_v, [jnp.zeros((L,), jnp.int32) + i])

        def col_body(c, carry, li=li, i=i, ci=ci):
            s = c * L
            off = li * N + s
            e_bit = _bit(ep_v[pl.ds(off, L)], u0_v[pl.ds(off, L)],
                         u1_v[pl.ds(off, L)])
            et_bit = _bit(ept_v[pl.ds(off, L)], u0t_v[pl.ds(off, L)],
                          u1t_v[pl.ds(off, L)])
            m = e_bit * croot_v[pl.ds(s, L)]
            mt = et_bit * ci
            jv = lanes + s
            keep = jnp.float32(1.0) - jnp.where(jv < i, mt, jnp.float32(0.0))
            dag = jnp.where(jv == i, jnp.float32(0.0), m * keep)
            out_v[pl.ds(off, L)] = dag
            return carry

        lax.fori_loop(0, NCOL, col_body, 0)

    pltpu.sync_copy(out_v, out_h.at[pl.ds(base, CHUNK)])


_sc_call = functools.partial(
    pl.kernel,
    out_type=jax.ShapeDtypeStruct((N * N,), jnp.float32),
    mesh=plsc.VectorSubcoreMesh(core_axis_name="c", subcore_axis_name="s"),
    scratch_types=[
        pltpu.VMEM((N,), jnp.float32),
        pltpu.VMEM((N,), jnp.float32),
        pltpu.VMEM((N,), jnp.float32),
        pltpu.VMEM((N,), jnp.float32),
        pltpu.VMEM((CHUNK,), jnp.float32),
        pltpu.VMEM((CHUNK,), jnp.float32),
        pltpu.VMEM((CHUNK,), jnp.float32),
        pltpu.VMEM((CHUNK,), jnp.float32),
        pltpu.VMEM((CHUNK,), jnp.float32),
        pltpu.VMEM((CHUNK,), jnp.float32),
        pltpu.VMEM((CHUNK,), jnp.float32),
    ],
)(_body)


def kernel(root_probs, edge_probs, u_root, u_edge):
    u0 = u_edge[:, :, 0]
    u1 = u_edge[:, :, 1]
    out = _sc_call(
        root_probs,
        u_root[:, 0], u_root[:, 1],
        edge_probs.reshape(-1),
        u0.reshape(-1), u1.reshape(-1),
        edge_probs.T.reshape(-1),
        u0.T.reshape(-1), u1.T.reshape(-1),
    )
    return out.reshape(N, N)


# speedup vs baseline: 26.9592x; 26.9592x over previous
import functools

import jax
import jax.numpy as jnp
from jax import lax
from jax.experimental import pallas as pl
from jax.experimental.pallas import tpu as pltpu
from jax.experimental.pallas import tpu_sc as plsc

N = 256
_INFO = plsc.get_sparse_core_info()
NC, NS, L = _INFO.num_cores, _INFO.num_subcores, _INFO.num_lanes
NW = NC * NS
RPW = N // NW
CHUNK = RPW * N
NCOL = N // L

_SQRT2 = 1.4142135623730951
_C1 = 2.885390081777927
_C3 = 0.9617966939259756
_C5 = 0.5770780163555854
_C7 = 0.4121985945111324
_C9 = 0.3205988979753252


def _log2(u):
    bits = lax.bitcast_convert_type(u, jnp.int32)
    e = (bits >> 23) - 127
    m = lax.bitcast_convert_type((bits & 0x7FFFFF) | 0x3F800000, jnp.float32)
    big = m >= jnp.float32(_SQRT2)
    m2 = jnp.where(big, m * jnp.float32(0.5), m)
    ef = e.astype(jnp.float32) + jnp.where(big, jnp.float32(1.0), jnp.float32(0.0))
    z = (m2 - jnp.float32(1.0)) / (m2 + jnp.float32(1.0))
    z2 = z * z
    pz = z * (jnp.float32(_C1) + z2 * (jnp.float32(_C3) + z2 * (
        jnp.float32(_C5) + z2 * (jnp.float32(_C7) + z2 * jnp.float32(_C9)))))
    return ef + pz


def _bit(p, u0, u1):
    w0 = -_log2(u0)
    w1 = -_log2(u1)
    t = jnp.exp(jnp.float32(2.0) * p - jnp.float32(1.0))
    return jnp.where(w1 * t >= w0, jnp.float32(1.0), jnp.float32(0.0))


def _body(rp_h, ur0_h, ur1_h, ep_h, u0_h, u1_h, ept_h, u0t_h, u1t_h, out_h,
          rp_v, ur0_v, ur1_v, croot_v,
          ep_v, u0_v, u1_v, ept_v, u0t_v, u1t_v, out_v,
          idx_v, g0_v, g1_v, g2_v, sem):
    wid = lax.axis_index("s") * NC + lax.axis_index("c")
    base = wid * CHUNK

    pltpu.sync_copy(rp_h, rp_v)
    pltpu.sync_copy(ur0_h, ur0_v)
    pltpu.sync_copy(ur1_h, ur1_v)
    pltpu.sync_copy(ep_h.at[pl.ds(base, CHUNK)], ep_v)
    pltpu.sync_copy(u0_h.at[pl.ds(base, CHUNK)], u0_v)
    pltpu.sync_copy(u1_h.at[pl.ds(base, CHUNK)], u1_v)
    pltpu.sync_copy(ept_h.at[pl.ds(base, CHUNK)], ept_v)
    pltpu.sync_copy(u0t_h.at[pl.ds(base, CHUNK)], u0t_v)
    pltpu.sync_copy(u1t_h.at[pl.ds(base, CHUNK)], u1t_v)

    def croot_body(c, carry):
        s = c * L
        bit = _bit(rp_v[pl.ds(s, L)], ur0_v[pl.ds(s, L)], ur1_v[pl.ds(s, L)])
        croot_v[pl.ds(s, L)] = jnp.float32(1.0) - bit
        return carry

    lax.fori_loop(0, NCOL, croot_body, 0)

    lanes = lax.iota(jnp.int32, L)
    for li in range(RPW):
        i = wid * RPW + li
        iv = jnp.zeros((L,), jnp.int32) + i
        idx_v[...] = iv
        pltpu.async_copy(rp_h.at[idx_v], g0_v, sem).wait()
        pltpu.async_copy(ur0_h.at[idx_v], g1_v, sem).wait()
        pltpu.async_copy(ur1_h.at[idx_v], g2_v, sem).wait()
        ci = jnp.float32(1.0) - _bit(g0_v[...], g1_v[...], g2_v[...])

        def col_body(c, carry, li=li, i=i, ci=ci):
            s = c * L
            off = li * N + s
            e_bit = _bit(ep_v[pl.ds(off, L)], u0_v[pl.ds(off, L)],
                         u1_v[pl.ds(off, L)])
            et_bit = _bit(ept_v[pl.ds(off, L)], u0t_v[pl.ds(off, L)],
                          u1t_v[pl.ds(off, L)])
            m = e_bit * croot_v[pl.ds(s, L)]
            mt = et_bit * ci
            jv = lanes + s
            keep = jnp.float32(1.0) - jnp.where(jv < i, mt, jnp.float32(0.0))
            dag = jnp.where(jv == i, jnp.float32(0.0), m * keep)
            out_v[pl.ds(off, L)] = dag
            return carry

        lax.fori_loop(0, NCOL, col_body, 0)

    pltpu.sync_copy(out_v, out_h.at[pl.ds(base, CHUNK)])


_sc_call = functools.partial(
    pl.kernel,
    out_type=jax.ShapeDtypeStruct((N * N,), jnp.float32),
    mesh=plsc.VectorSubcoreMesh(core_axis_name="c", subcore_axis_name="s"),
    scratch_types=[
        pltpu.VMEM((N,), jnp.float32),
        pltpu.VMEM((N,), jnp.float32),
        pltpu.VMEM((N,), jnp.float32),
        pltpu.VMEM((N,), jnp.float32),
        pltpu.VMEM((CHUNK,), jnp.float32),
        pltpu.VMEM((CHUNK,), jnp.float32),
        pltpu.VMEM((CHUNK,), jnp.float32),
        pltpu.VMEM((CHUNK,), jnp.float32),
        pltpu.VMEM((CHUNK,), jnp.float32),
        pltpu.VMEM((CHUNK,), jnp.float32),
        pltpu.VMEM((CHUNK,), jnp.float32),
        pltpu.VMEM((L,), jnp.int32),
        pltpu.VMEM((L,), jnp.float32),
        pltpu.VMEM((L,), jnp.float32),
        pltpu.VMEM((L,), jnp.float32),
        pltpu.SemaphoreType.DMA,
    ],
)(_body)


def kernel(root_probs, edge_probs, u_root, u_edge):
    u0 = u_edge[:, :, 0]
    u1 = u_edge[:, :, 1]
    out = _sc_call(
        root_probs,
        u_root[:, 0], u_root[:, 1],
        edge_probs.reshape(-1),
        u0.reshape(-1), u1.reshape(-1),
        edge_probs.T.reshape(-1),
        u0.T.reshape(-1), u1.T.reshape(-1),
    )
    return out.reshape(N, N)
